# baseline (speedup 1.0000x reference)
import numpy as np
import jax
import jax.numpy as jnp
from jax.experimental import pallas as pl
from jax.experimental.pallas import tpu as pltpu

_NF, _C, _NCODE = 63, 512, 512
_B, _T, _TQ = 32, 1024, 128
_F32 = jnp.float32
_HI = jax.lax.Precision.HIGHEST
_DP = jax.lax.Precision.HIGHEST

_PARAMS = pltpu.CompilerParams(
    dimension_semantics=("parallel",),
    vmem_limit_bytes=56 * 1024 * 1024,
)


def _interp_matrix() -> np.ndarray:
    t_in, out_len = _TQ, _T
    pos = (np.arange(out_len) + 0.5) * (t_in / out_len) - 0.5
    pos = np.clip(pos, 0.0, t_in - 1.0)
    i0 = np.floor(pos).astype(np.int32)
    i1 = np.minimum(i0 + 1, t_in - 1)
    w = (pos - i0).astype(np.float32)
    a = np.zeros((out_len, t_in), np.float32)
    a[np.arange(out_len), i0] += 1.0 - w
    a[np.arange(out_len), i1] += w
    return a


_INTERP_A = _interp_matrix()


def _shift(x, o):
    if o == 0:
        return x
    z = jnp.zeros((abs(o), x.shape[1]), x.dtype)
    if o > 0:
        return jnp.concatenate([x[o:], z], axis=0)
    return jnp.concatenate([z, x[:o]], axis=0)


def _conv3(x, w3, b, d=1):
    y = jnp.dot(_shift(x, -d), w3[0], preferred_element_type=_F32, precision=_DP)
    y = y + jnp.dot(x, w3[1], preferred_element_type=_F32, precision=_DP)
    y = y + jnp.dot(_shift(x, d), w3[2], preferred_element_type=_F32, precision=_DP)
    return y + b


def _resnet(x, rw1, rb1, rw2, rb2, reverse):
    dils = (9, 3, 1) if reverse else (1, 3, 9)
    for i, d in enumerate(dils):
        h = jax.nn.relu(x)
        h = _conv3(h, rw1.at[i], rb1[i], d)
        h = jax.nn.relu(h)
        h = jnp.dot(h, rw2[i], preferred_element_type=_F32, precision=_DP) + rb2[i]
        x = x + h
    return x


def _down2(h, s_refs, wd, bd):
    for i, s in enumerate(s_refs):
        s[...] = h[:, 128 * i:128 * (i + 1)]
    even = jnp.concatenate([s[0::2, :] for s in s_refs], axis=1)
    odd = jnp.concatenate([s[1::2, :] for s in s_refs], axis=1)
    y = jnp.dot(_shift(odd, -1), wd[0], preferred_element_type=_F32, precision=_DP)
    y = y + jnp.dot(even, wd[1], preferred_element_type=_F32, precision=_DP)
    y = y + jnp.dot(odd, wd[2], preferred_element_type=_F32, precision=_DP)
    return y + bd


def _enc_in_kernel(x_ref, wi_ref, bi_ref, wd_ref, bd_ref, o_ref, *s_refs):
    h = jax.nn.relu(_conv3(x_ref[...], wi_ref, bi_ref[...], 1))
    o_ref[...] = _down2(h, s_refs, wd_ref, bd_ref[...])


def _res_down_kernel(x_ref, rw1, rb1, rw2, rb2, wd_ref, bd_ref, o_ref,
                     *s_refs):
    x = _resnet(x_ref[...], rw1, rb1, rw2, rb2, reverse=False)
    o_ref[...] = _down2(x, s_refs, wd_ref, bd_ref[...])


def _res_out_kernel(x_ref, rw1, rb1, rw2, rb2, wo_ref, bo_ref, o_ref):
    x = _resnet(x_ref[...], rw1, rb1, rw2, rb2, reverse=False)
    o_ref[...] = _conv3(x, wo_ref, bo_ref[...], 1)


def _vq_kernel(xe_ref, cbt_ref, cb_ref, cbsq_ref, xq_ref, cnt_ref, cl_ref):
    xe = xe_ref[...]
    scores = jnp.dot(xe, cbt_ref[...], preferred_element_type=_F32, precision=_DP)
    d = cbsq_ref[...] - 2.0 * scores
    idx = jnp.argmin(d, axis=1, keepdims=True)
    onehot = (jax.lax.broadcasted_iota(jnp.int32, (_TQ, _NCODE), 1)
              == idx).astype(_F32)
    xd = jax.lax.dot_general(onehot, cb_ref[...], (((1,), (0,)), ((), ())),
                             precision=_HI)
    diff = xe - xd
    cl = jnp.sum(jnp.sum(diff * diff, axis=0, keepdims=True),
                 axis=1, keepdims=True)
    cl_ref[...] = jnp.broadcast_to(cl, (1, 8))
    cnt_ref[...] = jnp.sum(onehot, axis=0, keepdims=True)
    nrm_e = jnp.sqrt(jnp.sum(xe * xe, axis=0, keepdims=True))
    nrm_q = jnp.sqrt(jnp.sum(xd * xd, axis=0, keepdims=True))
    e_n = xe / jnp.maximum(nrm_e, 1e-12)
    q_n = xd / jnp.maximum(nrm_q, 1e-12)
    r = e_n + q_n
    r = r / jnp.maximum(jnp.sqrt(jnp.sum(r * r, axis=0, keepdims=True)), 1e-12)
    rx = jnp.sum(r * xe, axis=0, keepdims=True)
    ex = jnp.sum(e_n * xe, axis=0, keepdims=True)
    scaling = nrm_q / nrm_e
    xq_ref[...] = scaling * (xe - 2.0 * r * rx + 2.0 * q_n * ex)


def _stats_kernel(cl_ref, cnt_ref, loss_ref, perp_ref):
    s = jnp.sum(cl_ref[...][:, 0:1], axis=0, keepdims=True)
    loss_ref[...] = s * (1.0 / (_B * _TQ * _C))
    counts = jnp.sum(cnt_ref[...], axis=0, keepdims=True)
    p = counts * (1.0 / (_B * _TQ))
    ent = jnp.sum(p * jnp.log(p + 1e-10), axis=1, keepdims=True)
    perp_ref[...] = jnp.exp(-ent)


def _dec_in_kernel(x_ref, wi_ref, bi_ref, rw1, rb1, rw2, rb2, wu_ref, bu_ref,
                   o_ref):
    x = jax.nn.relu(_conv3(x_ref[...], wi_ref, bi_ref[...], 1))
    x = _resnet(x, rw1, rb1, rw2, rb2, reverse=True)
    o_ref[...] = _conv3(x, wu_ref, bu_ref[...], 1)


def _up_kernel(x_ref, rw1, rb1, rw2, rb2, wu_ref, bu_ref, o_ref):
    x = _resnet(x_ref[...], rw1, rb1, rw2, rb2, reverse=True)
    o_ref[...] = _conv3(x, wu_ref, bu_ref[...], 1)


def _dec_out_kernel(x_ref, rw1, rb1, rw2, rb2, wu_ref, bu_ref, wm_ref, bm_ref,
                    wo_ref, bo_ref, a_ref, o_ref):
    x = _resnet(x_ref[...], rw1, rb1, rw2, rb2, reverse=True)
    x = _conv3(x, wu_ref, bu_ref[...], 1)
    x = jax.nn.relu(_conv3(x, wm_ref, bm_ref[...], 1))
    y = _conv3(x, wo_ref, bo_ref[...], 1)
    o_ref[...] = jax.lax.dot_general(a_ref[...], y, (((1,), (0,)), ((), ())),
                                     precision=_HI)


def _bspec(*dims):
    n = len(dims)
    return pl.BlockSpec((None,) + dims, lambda b, _n=n: (b,) + (0,) * _n)


def _sspec(*dims):
    n = len(dims)
    return pl.BlockSpec(dims, lambda b, _n=n: (0,) * _n)


def _t3(w):
    return jnp.transpose(w, (2, 1, 0))


def _res_stack(blocks):
    rw1 = jnp.stack([_t3(p['w1']) for p in blocks])
    rb1 = jnp.stack([p['b1'][None, :] for p in blocks])
    rw2 = jnp.stack([jnp.transpose(p['w2'][:, :, 0]) for p in blocks])
    rb2 = jnp.stack([p['b2'][None, :] for p in blocks])
    return rw1, rb1, rw2, rb2


_RES_SPECS = [_sspec(3, 3, _C, _C), _sspec(3, 1, _C), _sspec(3, _C, _C),
              _sspec(3, 1, _C)]


def kernel(features, params):
    enc, dec, cb = params['enc'], params['dec'], params['codebook']

    d0, d1, d2 = enc['down']
    x = pl.pallas_call(
        _enc_in_kernel, grid=(_B,),
        in_specs=[_bspec(_T, _NF), _sspec(3, _NF, _C), _sspec(1, _C),
                  _sspec(3, _C, _C), _sspec(1, _C)],
        out_specs=_bspec(_T // 2, _C),
        out_shape=jax.ShapeDtypeStruct((_B, _T // 2, _C), _F32),
        scratch_shapes=[pltpu.VMEM((_T, 128), _F32)] * 4,
        compiler_params=_PARAMS,
    )(features, _t3(enc['in_w']), enc['in_b'][None, :],
      _t3(d0['w']), d0['b'][None, :])

    for tcur, blk_res, blk_next in ((_T // 2, d0, d1), (_T // 4, d1, d2)):
        x = pl.pallas_call(
            _res_down_kernel, grid=(_B,),
            in_specs=[_bspec(tcur, _C)] + _RES_SPECS
                     + [_sspec(3, _C, _C), _sspec(1, _C)],
            out_specs=_bspec(tcur // 2, _C),
            out_shape=jax.ShapeDtypeStruct((_B, tcur // 2, _C), _F32),
            scratch_shapes=[pltpu.VMEM((tcur, 128), _F32)] * 4,
            compiler_params=_PARAMS,
        )(x, *_res_stack(blk_res['res']), _t3(blk_next['w']),
          blk_next['b'][None, :])

    xe = pl.pallas_call(
        _res_out_kernel, grid=(_B,),
        in_specs=[_bspec(_TQ, _C)] + _RES_SPECS
                 + [_sspec(3, _C, _C), _sspec(1, _C)],
        out_specs=_bspec(_TQ, _C),
        out_shape=jax.ShapeDtypeStruct((_B, _TQ, _C), _F32),
        compiler_params=_PARAMS,
    )(x, *_res_stack(d2['res']), _t3(enc['out_w']), enc['out_b'][None, :])

    xq, cnts, cls = pl.pallas_call(
        _vq_kernel, grid=(_B,),
        in_specs=[_bspec(_TQ, _C), _sspec(_C, _NCODE), _sspec(_NCODE, _C),
                  _sspec(1, _NCODE)],
        out_specs=[_bspec(_TQ, _C), _bspec(1, _NCODE), _bspec(1, 8)],
        out_shape=[jax.ShapeDtypeStruct((_B, _TQ, _C), _F32),
                   jax.ShapeDtypeStruct((_B, 1, _NCODE), _F32),
                   jax.ShapeDtypeStruct((_B, 1, 8), _F32)],
        compiler_params=_PARAMS,
    )(xe, jnp.transpose(cb), cb, jnp.sum(cb * cb, axis=1)[None, :])

    loss, perp = pl.pallas_call(
        _stats_kernel,
        out_shape=[jax.ShapeDtypeStruct((1, 1), _F32),
                   jax.ShapeDtypeStruct((1, 1), _F32)],
    )(cls.reshape(_B, 8), cnts.reshape(_B, _NCODE))

    u0, u1, u2 = dec['up']
    x = pl.pallas_call(
        _dec_in_kernel, grid=(_B,),
        in_specs=[_bspec(_TQ, _C), _sspec(3, _C, _C), _sspec(1, _C)]
                 + _RES_SPECS + [_sspec(3, _C, _C), _sspec(1, _C)],
        out_specs=_bspec(_TQ, _C),
        out_shape=jax.ShapeDtypeStruct((_B, _TQ, _C), _F32),
        compiler_params=_PARAMS,
    )(xq, _t3(dec['in_w']), dec['in_b'][None, :], *_res_stack(u0['res']),
      _t3(u0['w']), u0['b'][None, :])

    x = pl.pallas_call(
        _up_kernel, grid=(_B,),
        in_specs=[_bspec(_TQ, _C)] + _RES_SPECS
                 + [_sspec(3, _C, _C), _sspec(1, _C)],
        out_specs=_bspec(_TQ, _C),
        out_shape=jax.ShapeDtypeStruct((_B, _TQ, _C), _F32),
        compiler_params=_PARAMS,
    )(x, *_res_stack(u1['res']), _t3(u1['w']), u1['b'][None, :])

    out = pl.pallas_call(
        _dec_out_kernel, grid=(_B,),
        in_specs=[_bspec(_TQ, _C)] + _RES_SPECS
                 + [_sspec(3, _C, _C), _sspec(1, _C), _sspec(3, _C, _C),
                    _sspec(1, _C), _sspec(3, _C, _NF), _sspec(1, _NF),
                    _sspec(_T, _TQ)],
        out_specs=_bspec(_T, _NF),
        out_shape=jax.ShapeDtypeStruct((_B, _T, _NF), _F32),
        compiler_params=_PARAMS,
    )(x, *_res_stack(u2['res']), _t3(u2['w']), u2['b'][None, :],
      _t3(dec['mid_w']), dec['mid_b'][None, :], _t3(dec['out_w']),
      dec['out_b'][None, :], jnp.asarray(_INTERP_A))

    return out, loss[0, 0], perp[0, 0]


# speedup vs baseline: 1.1710x; 1.1710x over previous
import numpy as np
import jax
import jax.numpy as jnp
from jax.experimental import pallas as pl
from jax.experimental.pallas import tpu as pltpu

_NF, _C, _NCODE = 63, 512, 512
_B, _T, _TQ = 32, 1024, 128
_F32 = jnp.float32
_BF16 = jnp.bfloat16
_HI = jax.lax.Precision.HIGHEST

_PARAMS = pltpu.CompilerParams(
    dimension_semantics=("parallel",),
    vmem_limit_bytes=56 * 1024 * 1024,
)


def _interp_matrix() -> np.ndarray:
    t_in, out_len = _TQ, _T
    pos = (np.arange(out_len) + 0.5) * (t_in / out_len) - 0.5
    pos = np.clip(pos, 0.0, t_in - 1.0)
    i0 = np.floor(pos).astype(np.int32)
    i1 = np.minimum(i0 + 1, t_in - 1)
    w = (pos - i0).astype(np.float32)
    a = np.zeros((out_len, t_in), np.float32)
    a[np.arange(out_len), i0] += 1.0 - w
    a[np.arange(out_len), i1] += w
    return a


_INTERP_A = _interp_matrix()


def _split_n(x, n):
    parts = []
    r = x
    for i in range(n):
        p = r.astype(_BF16)
        parts.append(p)
        if i < n - 1:
            r = r - p.astype(_F32)
    return parts


def _mmn(xp, wp):
    n = len(xp)
    y = None
    for i in range(n):
        for j in range(n - i):
            t = jnp.dot(xp[i], wp[j], preferred_element_type=_F32)
            y = t if y is None else y + t
    return y


def _shift(x, o):
    if o == 0:
        return x
    z = jnp.zeros((abs(o), x.shape[1]), x.dtype)
    if o > 0:
        return jnp.concatenate([x[o:], z], axis=0)
    return jnp.concatenate([z, x[:o]], axis=0)


def _conv3(x, wparts, b, d=1):
    xp = _split_n(x, len(wparts))
    q0 = _mmn(xp, [w[0] for w in wparts])
    q1 = _mmn(xp, [w[1] for w in wparts])
    q2 = _mmn(xp, [w[2] for w in wparts])
    return _shift(q0, -d) + q1 + _shift(q2, d) + b


def _resnet(x, rw1, rb1, rw2, rb2, reverse):
    dils = (9, 3, 1) if reverse else (1, 3, 9)
    for i, d in enumerate(dils):
        h = jax.nn.relu(x)
        h = _conv3(h, tuple(w.at[i] for w in rw1), rb1[i], d)
        h = jax.nn.relu(h)
        hp = _split_n(h, len(rw2))
        h = _mmn(hp, [w[i] for w in rw2]) + rb2[i]
        x = x + h
    return x


def _down2(h, s_refs, wparts, bd):
    for i, s in enumerate(s_refs):
        s[...] = h[:, 128 * i:128 * (i + 1)]
    even = jnp.concatenate([s[0::2, :] for s in s_refs], axis=1)
    odd = jnp.concatenate([s[1::2, :] for s in s_refs], axis=1)
    ep = _split_n(even, len(wparts))
    op = _split_n(odd, len(wparts))
    y = _shift(_mmn(op, [w[0] for w in wparts]), -1)
    y = y + _mmn(ep, [w[1] for w in wparts])
    y = y + _mmn(op, [w[2] for w in wparts])
    return y + bd


def _enc_in_kernel(x_ref, wi1, wi2, wi3, bi_ref, wd1, wd2, wd3, bd_ref,
                   o_ref, *s_refs):
    h = jax.nn.relu(_conv3(x_ref[...], (wi1, wi2, wi3), bi_ref[...], 1))
    o_ref[...] = _down2(h, s_refs, (wd1, wd2, wd3), bd_ref[...])


def _res_down_kernel(x_ref, ra1, ra2, ra3, rb1, rc1, rc2, rc3, rb2,
                     wd1, wd2, wd3, bd_ref, o_ref, *s_refs):
    x = _resnet(x_ref[...], (ra1, ra2, ra3), rb1, (rc1, rc2, rc3), rb2,
                reverse=False)
    o_ref[...] = _down2(x, s_refs, (wd1, wd2, wd3), bd_ref[...])


def _res_out_kernel(x_ref, ra1, ra2, ra3, rb1, rc1, rc2, rc3, rb2,
                    wo1, wo2, wo3, bo_ref, o_ref):
    x = _resnet(x_ref[...], (ra1, ra2, ra3), rb1, (rc1, rc2, rc3), rb2,
                reverse=False)
    o_ref[...] = _conv3(x, (wo1, wo2, wo3), bo_ref[...], 1)


def _vq_kernel(xe_ref, ct1_ref, ct2_ref, ct3_ref, cb_ref, cbsq_ref, xq_ref,
               cnt_ref, cl_ref):
    xe = xe_ref[...]
    xp = _split_n(xe, 3)
    scores = _mmn(xp, [ct1_ref[...], ct2_ref[...], ct3_ref[...]])
    xfsq = jnp.sum(xe * xe, axis=1, keepdims=True)
    d = xfsq - 2.0 * scores + cbsq_ref[...]
    idx = jnp.argmin(d, axis=1, keepdims=True)
    onehot = (jax.lax.broadcasted_iota(jnp.int32, (_TQ, _NCODE), 1)
              == idx).astype(_F32)
    xd = jax.lax.dot_general(onehot, cb_ref[...], (((1,), (0,)), ((), ())),
                             precision=_HI)
    diff = xe - xd
    cl = jnp.sum(jnp.sum(diff * diff, axis=0, keepdims=True),
                 axis=1, keepdims=True)
    cl_ref[...] = jnp.broadcast_to(cl, (1, 8))
    cnt_ref[...] = jnp.sum(onehot, axis=0, keepdims=True)
    nrm_e = jnp.sqrt(jnp.sum(xe * xe, axis=0, keepdims=True))
    nrm_q = jnp.sqrt(jnp.sum(xd * xd, axis=0, keepdims=True))
    e_n = xe / jnp.maximum(nrm_e, 1e-12)
    q_n = xd / jnp.maximum(nrm_q, 1e-12)
    r = e_n + q_n
    r = r / jnp.maximum(jnp.sqrt(jnp.sum(r * r, axis=0, keepdims=True)), 1e-12)
    rx = jnp.sum(r * xe, axis=0, keepdims=True)
    ex = jnp.sum(e_n * xe, axis=0, keepdims=True)
    scaling = nrm_q / nrm_e
    xq_ref[...] = scaling * (xe - 2.0 * r * rx + 2.0 * q_n * ex)


def _stats_kernel(cl_ref, cnt_ref, loss_ref, perp_ref):
    s = jnp.sum(cl_ref[...][:, 0:1], axis=0, keepdims=True)
    loss_ref[...] = s * (1.0 / (_B * _TQ * _C))
    counts = jnp.sum(cnt_ref[...], axis=0, keepdims=True)
    p = counts * (1.0 / (_B * _TQ))
    ent = jnp.sum(p * jnp.log(p + 1e-10), axis=1, keepdims=True)
    perp_ref[...] = jnp.exp(-ent)


def _dec_in_kernel(x_ref, wi1, wi2, bi_ref, ra1, ra2, rb1, rc1, rc2, rb2,
                   wu1, wu2, bu_ref, o_ref):
    x = jax.nn.relu(_conv3(x_ref[...], (wi1, wi2), bi_ref[...], 1))
    x = _resnet(x, (ra1, ra2), rb1, (rc1, rc2), rb2, reverse=True)
    o_ref[...] = _conv3(x, (wu1, wu2), bu_ref[...], 1)


def _up_kernel(x_ref, ra1, ra2, rb1, rc1, rc2, rb2, wu1, wu2, bu_ref, o_ref):
    x = _resnet(x_ref[...], (ra1, ra2), rb1, (rc1, rc2), rb2, reverse=True)
    o_ref[...] = _conv3(x, (wu1, wu2), bu_ref[...], 1)


def _dec_out_kernel(x_ref, ra1, ra2, rb1, rc1, rc2, rb2, wu1, wu2, bu_ref,
                    wm1, wm2, bm_ref, wo1, wo2, bo_ref, a_ref, o_ref):
    x = _resnet(x_ref[...], (ra1, ra2), rb1, (rc1, rc2), rb2, reverse=True)
    x = _conv3(x, (wu1, wu2), bu_ref[...], 1)
    x = jax.nn.relu(_conv3(x, (wm1, wm2), bm_ref[...], 1))
    y = _conv3(x, (wo1, wo2), bo_ref[...], 1)
    o_ref[...] = jax.lax.dot_general(a_ref[...], y, (((1,), (0,)), ((), ())),
                                     precision=_HI)


def _bspec(*dims):
    n = len(dims)
    return pl.BlockSpec((None,) + dims, lambda b, _n=n: (b,) + (0,) * _n)


def _sspec(*dims):
    n = len(dims)
    return pl.BlockSpec(dims, lambda b, _n=n: (0,) * _n)


def _t3(w):
    return jnp.transpose(w, (2, 1, 0))


def _wsplit_n(w, n):
    parts = []
    r = w
    for i in range(n):
        p = r.astype(_BF16)
        parts.append(p)
        if i < n - 1:
            r = r - p.astype(_F32)
    return tuple(parts)


def _conv_w(w, n):
    return _wsplit_n(_t3(w), n)


def _res_stack(blocks, n):
    w1p = _wsplit_n(jnp.stack([_t3(p['w1']) for p in blocks]), n)
    rb1 = jnp.stack([p['b1'][None, :] for p in blocks])
    w2p = _wsplit_n(jnp.stack([jnp.transpose(p['w2'][:, :, 0])
                               for p in blocks]), n)
    rb2 = jnp.stack([p['b2'][None, :] for p in blocks])
    return w1p + (rb1,) + w2p + (rb2,)


def _res_specs(n):
    return ([_sspec(3, 3, _C, _C)] * n + [_sspec(3, 1, _C)]
            + [_sspec(3, _C, _C)] * n + [_sspec(3, 1, _C)])


def kernel(features, params):
    enc, dec, cb = params['enc'], params['dec'], params['codebook']

    d0, d1, d2 = enc['down']
    x = pl.pallas_call(
        _enc_in_kernel, grid=(_B,),
        in_specs=[_bspec(_T, _NF)] + [_sspec(3, _NF, _C)] * 3
                 + [_sspec(1, _C)] + [_sspec(3, _C, _C)] * 3 + [_sspec(1, _C)],
        out_specs=_bspec(_T // 2, _C),
        out_shape=jax.ShapeDtypeStruct((_B, _T // 2, _C), _F32),
        scratch_shapes=[pltpu.VMEM((_T, 128), _F32)] * 4,
        compiler_params=_PARAMS,
    )(features, *_conv_w(enc['in_w'], 3), enc['in_b'][None, :],
      *_conv_w(d0['w'], 3), d0['b'][None, :])

    for tcur, blk_res, blk_next in ((_T // 2, d0, d1), (_T // 4, d1, d2)):
        x = pl.pallas_call(
            _res_down_kernel, grid=(_B,),
            in_specs=[_bspec(tcur, _C)] + _res_specs(3)
                     + [_sspec(3, _C, _C)] * 3 + [_sspec(1, _C)],
            out_specs=_bspec(tcur // 2, _C),
            out_shape=jax.ShapeDtypeStruct((_B, tcur // 2, _C), _F32),
            scratch_shapes=[pltpu.VMEM((tcur, 128), _F32)] * 4,
            compiler_params=_PARAMS,
        )(x, *_res_stack(blk_res['res'], 3), *_conv_w(blk_next['w'], 3),
          blk_next['b'][None, :])

    xe = pl.pallas_call(
        _res_out_kernel, grid=(_B,),
        in_specs=[_bspec(_TQ, _C)] + _res_specs(3)
                 + [_sspec(3, _C, _C)] * 3 + [_sspec(1, _C)],
        out_specs=_bspec(_TQ, _C),
        out_shape=jax.ShapeDtypeStruct((_B, _TQ, _C), _F32),
        compiler_params=_PARAMS,
    )(x, *_res_stack(d2['res'], 3), *_conv_w(enc['out_w'], 3),
      enc['out_b'][None, :])

    ctp = _wsplit_n(jnp.transpose(cb), 3)
    xq, cnts, cls = pl.pallas_call(
        _vq_kernel, grid=(_B,),
        in_specs=[_bspec(_TQ, _C)] + [_sspec(_C, _NCODE)] * 3
                 + [_sspec(_NCODE, _C), _sspec(1, _NCODE)],
        out_specs=[_bspec(_TQ, _C), _bspec(1, _NCODE), _bspec(1, 8)],
        out_shape=[jax.ShapeDtypeStruct((_B, _TQ, _C), _F32),
                   jax.ShapeDtypeStruct((_B, 1, _NCODE), _F32),
                   jax.ShapeDtypeStruct((_B, 1, 8), _F32)],
        compiler_params=_PARAMS,
    )(xe, *ctp, cb, jnp.sum(cb * cb, axis=1)[None, :])

    loss, perp = pl.pallas_call(
        _stats_kernel,
        out_shape=[jax.ShapeDtypeStruct((1, 1), _F32),
                   jax.ShapeDtypeStruct((1, 1), _F32)],
    )(cls.reshape(_B, 8), cnts.reshape(_B, _NCODE))

    u0, u1, u2 = dec['up']
    x = pl.pallas_call(
        _dec_in_kernel, grid=(_B,),
        in_specs=[_bspec(_TQ, _C)] + [_sspec(3, _C, _C)] * 2
                 + [_sspec(1, _C)] + _res_specs(2)
                 + [_sspec(3, _C, _C)] * 2 + [_sspec(1, _C)],
        out_specs=_bspec(_TQ, _C),
        out_shape=jax.ShapeDtypeStruct((_B, _TQ, _C), _F32),
        compiler_params=_PARAMS,
    )(xq, *_conv_w(dec['in_w'], 2), dec['in_b'][None, :],
      *_res_stack(u0['res'], 2), *_conv_w(u0['w'], 2), u0['b'][None, :])

    x = pl.pallas_call(
        _up_kernel, grid=(_B,),
        in_specs=[_bspec(_TQ, _C)] + _res_specs(2)
                 + [_sspec(3, _C, _C)] * 2 + [_sspec(1, _C)],
        out_specs=_bspec(_TQ, _C),
        out_shape=jax.ShapeDtypeStruct((_B, _TQ, _C), _F32),
        compiler_params=_PARAMS,
    )(x, *_res_stack(u1['res'], 2), *_conv_w(u1['w'], 2), u1['b'][None, :])

    out = pl.pallas_call(
        _dec_out_kernel, grid=(_B,),
        in_specs=[_bspec(_TQ, _C)] + _res_specs(2)
                 + [_sspec(3, _C, _C)] * 2 + [_sspec(1, _C)]
                 + [_sspec(3, _C, _C)] * 2 + [_sspec(1, _C)]
                 + [_sspec(3, _C, _NF)] * 2 + [_sspec(1, _NF)]
                 + [_sspec(_T, _TQ)],
        out_specs=_bspec(_T, _NF),
        out_shape=jax.ShapeDtypeStruct((_B, _T, _NF), _F32),
        compiler_params=_PARAMS,
    )(x, *_res_stack(u2['res'], 2), *_conv_w(u2['w'], 2), u2['b'][None, :],
      *_conv_w(dec['mid_w'], 2), dec['mid_b'][None, :],
      *_conv_w(dec['out_w'], 2), dec['out_b'][None, :],
      jnp.asarray(_INTERP_A))

    return out, loss[0, 0], perp[0, 0]
